# baseline (speedup 1.0000x reference)
import functools

import jax
import jax.numpy as jnp
from jax import lax
from jax.experimental import pallas as pl
from jax.experimental.pallas import tpu as pltpu
from jax.experimental.pallas import tpu_sc as plsc

N = 10000
E = 320000
DIN = 128
DHID = 128
NCLS = 64
EPS = 1e-5

TILES = 16
K = 128
S = 160
CH = 80
NCH = S // CH
EP = TILES * S * K
ZROWS = 632
NACC = TILES * ZROWS


def _make_spmm(C):
    mesh = plsc.VectorSubcoreMesh(core_axis_name="c", subcore_axis_name="s")
    TROWS = N // TILES

    @functools.partial(
        pl.kernel,
        out_type=jax.ShapeDtypeStruct((2, NACC, C), jnp.float32),
        mesh=mesh,
        compiler_params=pltpu.CompilerParams(use_tc_tiling_on_sc=False),
        scratch_types=[
            pltpu.VMEM_SHARED((NACC, C), jnp.float32),
            pltpu.VMEM_SHARED((N, C), jnp.float32),
            pltpu.VMEM((CH, K), jnp.int32),
            pltpu.VMEM((CH, K), jnp.int32),
            pltpu.VMEM((K, C), jnp.float32),
            pltpu.VMEM((K, C), jnp.float32),
            pltpu.SemaphoreType.DMA,
            pltpu.SemaphoreType.DMA,
        ],
    )
    def spmm(table_h, src_h, dst_h, zeros_h, out_h, acc, table_sh,
             src_v, dst_v, rows0, rows1, sem0, sem1):
        c = lax.axis_index("c")
        s = lax.axis_index("s")
        pltpu.sync_copy(table_h.at[c, pl.ds(s * TROWS, TROWS)],
                        table_sh.at[pl.ds(s * TROWS, TROWS)])
        pltpu.sync_copy(zeros_h.at[pl.ds(s * ZROWS, ZROWS)],
                        acc.at[pl.ds(s * ZROWS, ZROWS)])
        plsc.subcore_barrier()

        def chunk(ci, carry):
            pltpu.sync_copy(src_h.at[s, pl.ds(ci * CH, CH)], src_v)
            pltpu.sync_copy(dst_h.at[s, pl.ds(ci * CH, CH)], dst_v)

            def pair(p, carry2):
                j = 2 * p
                d0 = pltpu.async_copy(table_sh.at[src_v.at[j]], rows0, sem0)
                d1 = pltpu.async_copy(table_sh.at[src_v.at[j + 1]], rows1, sem1)
                d0.wait()
                pltpu.sync_copy(rows0, acc.at[dst_v.at[j]], add=True)
                d1.wait()
                pltpu.sync_copy(rows1, acc.at[dst_v.at[j + 1]], add=True)
                return carry2

            lax.fori_loop(0, CH // 2, pair, 0)
            return carry

        lax.fori_loop(0, NCH, chunk, 0)
        plsc.subcore_barrier()
        pltpu.sync_copy(acc.at[pl.ds(s * ZROWS, ZROWS)],
                        out_h.at[c, pl.ds(s * ZROWS, ZROWS)])

    return spmm


_spmm64 = _make_spmm(DHID // 2)
_spmm32 = _make_spmm(NCLS // 2)


def _mm_split_kernel(x_ref, w_ref, o_ref):
    o_ref[0] = jnp.dot(x_ref[...], w_ref[0],
                       preferred_element_type=jnp.float32)


def _mid_kernel(p_ref, g_ref, b_ref, w_ref, o_ref):
    def norm(part, g, b):
        hp = jnp.maximum(part, 0.0)
        m = jnp.mean(hp, axis=0, keepdims=True)
        v = jnp.mean((hp - m) ** 2, axis=0, keepdims=True)
        return jnp.maximum((hp - m) * lax.rsqrt(v + EPS) * g + b, 0.0)

    h0 = norm(p_ref[0], g_ref[0], b_ref[0])
    h1 = norm(p_ref[1], g_ref[1], b_ref[1])
    s2 = (jnp.dot(h0, w_ref[0], preferred_element_type=jnp.float32)
          + jnp.dot(h1, w_ref[1], preferred_element_type=jnp.float32))
    o_ref[0] = s2[:, :NCLS // 2]
    o_ref[1] = s2[:, NCLS // 2:]


def _out_kernel(p_ref, g_ref, b_ref, o_ref):
    def norm(part, g, b):
        hp = jnp.maximum(part, 0.0)
        m = jnp.mean(hp, axis=0, keepdims=True)
        v = jnp.mean((hp - m) ** 2, axis=0, keepdims=True)
        return (hp - m) * lax.rsqrt(v + EPS) * g + b

    o_ref[:, :NCLS // 2] = norm(p_ref[0], g_ref[0], b_ref[0])
    o_ref[:, NCLS // 2:] = norm(p_ref[1], g_ref[1], b_ref[1])


def kernel(x, edge_index, W1, gamma1, beta1, W2, gamma2, beta2):
    src = edge_index[0].astype(jnp.int32)
    dst = edge_index[1].astype(jnp.int32)
    pad = EP - E
    src_p = jnp.concatenate([src, jnp.zeros((pad,), jnp.int32)])
    dst_p = jnp.concatenate([dst, jnp.full((pad,), N, jnp.int32)])
    src_idx = src_p.reshape(TILES, S, K)
    dst_idx = dst_p.reshape(TILES, S, K)
    zeros64 = jnp.zeros((NACC, DHID // 2), jnp.float32)
    zeros32 = jnp.zeros((NACC, NCLS // 2), jnp.float32)

    support_parts = pl.pallas_call(
        _mm_split_kernel,
        grid=(2, 5),
        in_specs=[pl.BlockSpec((2000, DIN), lambda c, r: (r, 0)),
                  pl.BlockSpec((1, DIN, DHID // 2), lambda c, r: (c, 0, 0))],
        out_specs=pl.BlockSpec((1, 2000, DHID // 2), lambda c, r: (c, r, 0)),
        out_shape=jax.ShapeDtypeStruct((2, N, DHID // 2), jnp.float32),
    )(x, jnp.stack([W1[:, :DHID // 2], W1[:, DHID // 2:]]))

    h1_parts = _spmm64(support_parts, src_idx, dst_idx, zeros64)[:, :N, :]

    support2_parts = pl.pallas_call(
        _mid_kernel,
        out_shape=jax.ShapeDtypeStruct((2, N, NCLS // 2), jnp.float32),
    )(h1_parts,
      gamma1.reshape(2, 1, DHID // 2),
      beta1.reshape(2, 1, DHID // 2),
      W2.reshape(2, DHID // 2, NCLS))

    h2_parts = _spmm32(support2_parts, src_idx, dst_idx, zeros32)[:, :N, :]

    out = pl.pallas_call(
        _out_kernel,
        out_shape=jax.ShapeDtypeStruct((N, NCLS), jnp.float32),
    )(h2_parts,
      gamma2.reshape(2, 1, NCLS // 2),
      beta2.reshape(2, 1, NCLS // 2))
    return out


# speedup vs baseline: 1.1046x; 1.1046x over previous
import functools

import jax
import jax.numpy as jnp
from jax import lax
from jax.experimental import pallas as pl
from jax.experimental.pallas import tpu as pltpu
from jax.experimental.pallas import tpu_sc as plsc

N = 10000
E = 320000
DIN = 128
DHID = 128
NCLS = 64
EPS = 1e-5

TILES = 16
K = 128
S = 160
EP = TILES * S * K
ZROWS = 632
NACC = TILES * ZROWS


def _make_spmm(C):
    mesh = plsc.VectorSubcoreMesh(core_axis_name="c", subcore_axis_name="s")
    TROWS = N // TILES

    @functools.partial(
        pl.kernel,
        out_type=jax.ShapeDtypeStruct((2, NACC, C), jnp.float32),
        mesh=mesh,
        compiler_params=pltpu.CompilerParams(use_tc_tiling_on_sc=False),
        scratch_types=[
            pltpu.VMEM_SHARED((NACC, C), jnp.float32),
            pltpu.VMEM_SHARED((N, C), jnp.float32),
            pltpu.VMEM((S, K), jnp.int32),
            pltpu.VMEM((S, K), jnp.int32),
            pltpu.VMEM((K, C), jnp.float32),
            pltpu.SemaphoreType.DMA,
        ],
    )
    def spmm(table_h, src_h, dst_h, zeros_h, out_h, acc, table_sh,
             src_v, dst_v, rows_v, sem):
        c = lax.axis_index("c")
        s = lax.axis_index("s")
        p0 = pltpu.async_copy(src_h.at[s], src_v, sem)
        p1 = pltpu.async_copy(dst_h.at[s], dst_v, sem)
        p2 = pltpu.async_copy(table_h.at[c, pl.ds(s * TROWS, TROWS)],
                              table_sh.at[pl.ds(s * TROWS, TROWS)], sem)
        p3 = pltpu.async_copy(zeros_h.at[pl.ds(s * ZROWS, ZROWS)],
                              acc.at[pl.ds(s * ZROWS, ZROWS)], sem)
        p0.wait()
        p1.wait()
        p2.wait()
        p3.wait()
        plsc.subcore_barrier()

        def step(j, carry):
            pltpu.async_copy(table_sh.at[src_v.at[j]], rows_v, sem).wait()
            pltpu.sync_copy(rows_v, acc.at[dst_v.at[j]], add=True)
            return carry

        lax.fori_loop(0, S, step, 0)
        plsc.subcore_barrier()
        pltpu.sync_copy(acc.at[pl.ds(s * ZROWS, ZROWS)],
                        out_h.at[c, pl.ds(s * ZROWS, ZROWS)])

    return spmm


_spmm64 = _make_spmm(DHID // 2)


def _rsqrt16(v):
    half = v * 0.5
    i = plsc.bitcast(v, jnp.int32)
    y = plsc.bitcast(jnp.int32(0x5F3759DF) - (i >> 1), jnp.float32)
    for _ in range(3):
        y = y * (1.5 - half * y * y)
    return y


def _make_spmm_bn():
    C = NCLS // 2
    mesh = plsc.VectorSubcoreMesh(core_axis_name="c", subcore_axis_name="s")
    TROWS = N // TILES

    @functools.partial(
        pl.kernel,
        out_type=jax.ShapeDtypeStruct((N, NCLS), jnp.float32),
        mesh=mesh,
        compiler_params=pltpu.CompilerParams(use_tc_tiling_on_sc=False,
                                             needs_layout_passes=False),
        scratch_types=[
            pltpu.VMEM_SHARED((NACC, C), jnp.float32),
            pltpu.VMEM_SHARED((N, C), jnp.float32),
            pltpu.VMEM_SHARED((TILES, 4, 16), jnp.float32),
            pltpu.VMEM((S, K), jnp.int32),
            pltpu.VMEM((S, K), jnp.int32),
            pltpu.VMEM((K, C), jnp.float32),
            pltpu.VMEM((TROWS, C), jnp.float32),
            pltpu.VMEM((4, 16), jnp.float32),
            pltpu.VMEM((TILES, 4, 16), jnp.float32),
            pltpu.VMEM((C,), jnp.float32),
            pltpu.VMEM((C,), jnp.float32),
            pltpu.SemaphoreType.DMA,
        ],
    )
    def spmm_bn(table_h, src_h, dst_h, zeros_h, g_h, b_h, out_h,
                acc, table_sh, parts_sh,
                src_v, dst_v, rows_v, slab_v, mypart_v, parts_v,
                g_v, b_v, sem):
        c = lax.axis_index("c")
        s = lax.axis_index("s")
        p0 = pltpu.async_copy(src_h.at[s], src_v, sem)
        p1 = pltpu.async_copy(dst_h.at[s], dst_v, sem)
        p2 = pltpu.async_copy(g_h.at[pl.ds(c * C, C)], g_v, sem)
        p3 = pltpu.async_copy(b_h.at[pl.ds(c * C, C)], b_v, sem)
        p4 = pltpu.async_copy(table_h.at[c, pl.ds(s * TROWS, TROWS)],
                              table_sh.at[pl.ds(s * TROWS, TROWS)], sem)
        p5 = pltpu.async_copy(zeros_h.at[pl.ds(s * ZROWS, ZROWS)],
                              acc.at[pl.ds(s * ZROWS, ZROWS)], sem)
        p0.wait()
        p1.wait()
        p2.wait()
        p3.wait()
        p4.wait()
        p5.wait()
        plsc.subcore_barrier()

        def step(j, carry):
            pltpu.async_copy(table_sh.at[src_v.at[j]], rows_v, sem).wait()
            pltpu.sync_copy(rows_v, acc.at[dst_v.at[j]], add=True)
            return carry

        lax.fori_loop(0, S, step, 0)
        plsc.subcore_barrier()

        pltpu.sync_copy(acc.at[pl.ds(s * TROWS, TROWS)], slab_v)
        zv = jnp.zeros((16,), jnp.float32)

        def rbody(i, carry):
            slo, shi, qlo, qhi = carry
            lo = jnp.maximum(slab_v[i, pl.ds(0, 16)], 0.0)
            hi = jnp.maximum(slab_v[i, pl.ds(16, 16)], 0.0)
            slab_v[i, pl.ds(0, 16)] = lo
            slab_v[i, pl.ds(16, 16)] = hi
            return (slo + lo, shi + hi, qlo + lo * lo, qhi + hi * hi)

        slo, shi, qlo, qhi = lax.fori_loop(0, TROWS, rbody, (zv, zv, zv, zv))
        mypart_v[0] = slo
        mypart_v[1] = shi
        mypart_v[2] = qlo
        mypart_v[3] = qhi
        pltpu.sync_copy(mypart_v, parts_sh.at[s])
        plsc.subcore_barrier()
        pltpu.sync_copy(parts_sh, parts_v)

        def red(i, carry):
            a, b2, d, e = carry
            return (a + parts_v[i, 0], b2 + parts_v[i, 1],
                    d + parts_v[i, 2], e + parts_v[i, 3])

        tlo, thi, tqlo, tqhi = lax.fori_loop(0, TILES, red, (zv, zv, zv, zv))
        inv_n = jnp.float32(1.0 / N)
        mlo = tlo * inv_n
        mhi = thi * inv_n
        vlo = tqlo * inv_n - mlo * mlo
        vhi = tqhi * inv_n - mhi * mhi
        sc_lo = _rsqrt16(vlo + EPS) * g_v[pl.ds(0, 16)]
        sc_hi = _rsqrt16(vhi + EPS) * g_v[pl.ds(16, 16)]
        off_lo = b_v[pl.ds(0, 16)] - mlo * sc_lo
        off_hi = b_v[pl.ds(16, 16)] - mhi * sc_hi

        def nbody(i, carry):
            slab_v[i, pl.ds(0, 16)] = slab_v[i, pl.ds(0, 16)] * sc_lo + off_lo
            slab_v[i, pl.ds(16, 16)] = slab_v[i, pl.ds(16, 16)] * sc_hi + off_hi
            return carry

        lax.fori_loop(0, TROWS, nbody, 0)
        pltpu.sync_copy(slab_v,
                        out_h.at[pl.ds(s * TROWS, TROWS), pl.ds(c * C, C)])

    return spmm_bn


_spmm32_bn = _make_spmm_bn()


def _mm_split_kernel(x_ref, w_ref, o_ref):
    o_ref[0] = jnp.dot(x_ref[...], w_ref[0],
                       preferred_element_type=jnp.float32)


def _mid_kernel(p_ref, g_ref, b_ref, w_ref, o_ref):
    def norm(part, g, b):
        hp = jnp.maximum(part, 0.0)
        m = jnp.mean(hp, axis=0, keepdims=True)
        v = jnp.mean((hp - m) ** 2, axis=0, keepdims=True)
        return jnp.maximum((hp - m) * lax.rsqrt(v + EPS) * g + b, 0.0)

    h0 = norm(p_ref[0, :N], g_ref[0], b_ref[0])
    h1 = norm(p_ref[1, :N], g_ref[1], b_ref[1])
    s2 = (jnp.dot(h0, w_ref[0], preferred_element_type=jnp.float32)
          + jnp.dot(h1, w_ref[1], preferred_element_type=jnp.float32))
    o_ref[0] = s2[:, :NCLS // 2]
    o_ref[1] = s2[:, NCLS // 2:]


def kernel(x, edge_index, W1, gamma1, beta1, W2, gamma2, beta2):
    src = edge_index[0].astype(jnp.int32)
    dst = edge_index[1].astype(jnp.int32)
    pad = EP - E
    src_p = jnp.concatenate([src, jnp.zeros((pad,), jnp.int32)])
    dst_p = jnp.concatenate([dst, jnp.full((pad,), N, jnp.int32)])
    src_idx = src_p.reshape(TILES, S, K)
    dst_idx = dst_p.reshape(TILES, S, K)
    zeros64 = jnp.zeros((NACC, DHID // 2), jnp.float32)
    zeros32 = jnp.zeros((NACC, NCLS // 2), jnp.float32)

    support_parts = pl.pallas_call(
        _mm_split_kernel,
        grid=(2, 5),
        in_specs=[pl.BlockSpec((2000, DIN), lambda c, r: (r, 0)),
                  pl.BlockSpec((1, DIN, DHID // 2), lambda c, r: (c, 0, 0))],
        out_specs=pl.BlockSpec((1, 2000, DHID // 2), lambda c, r: (c, r, 0)),
        out_shape=jax.ShapeDtypeStruct((2, N, DHID // 2), jnp.float32),
    )(x, jnp.stack([W1[:, :DHID // 2], W1[:, DHID // 2:]]))

    h1_parts = _spmm64(support_parts, src_idx, dst_idx, zeros64)

    support2_parts = pl.pallas_call(
        _mid_kernel,
        out_shape=jax.ShapeDtypeStruct((2, N, NCLS // 2), jnp.float32),
    )(h1_parts,
      gamma1.reshape(2, 1, DHID // 2),
      beta1.reshape(2, 1, DHID // 2),
      W2.reshape(2, DHID // 2, NCLS))

    out = _spmm32_bn(support2_parts, src_idx, dst_idx, zeros32,
                     gamma2, beta2)
    return out
